# Initial kernel scaffold; baseline (speedup 1.0000x reference)
import jax
import jax.numpy as jnp
from jax.experimental import pallas as pl


def kernel(x, edge_index, batch, Wl1, Wr1, b1, Wl2, Wr2, b2, Wf1, bf1, Wf2, bf2):
    raise NotImplementedError("write your pallas kernel here")


# speedup vs baseline: 3.2048x; 3.2048x over previous
import functools

import jax
import jax.numpy as jnp
from jax import lax
from jax.experimental import pallas as pl
from jax.experimental.pallas import tpu as pltpu
from jax.experimental.pallas import tpu_sc as plsc

_N = 10000
_E = 160000
_D = 256
_G = 128
_f32 = jnp.float32

_BN = 2000
_NB = _N // _BN

_NSUB = 16
_CH = 80
_EP = _E // _NSUB
_NZ = 624
_NTAIL = _N - _NZ * _NSUB
_NTOFF = _NZ * _NSUB


def _tc_lin_body(x_ref, w_ref, b_ref, yl0_ref, yl1_ref, yrb_ref):
    r = jnp.dot(x_ref[...], w_ref[...], preferred_element_type=_f32)
    yl0_ref[...] = r[:, :128]
    yl1_ref[...] = r[:, 128:256]
    yrb_ref[...] = r[:, 256:] + b_ref[...][None, :]


_tc_lin = pl.pallas_call(
    _tc_lin_body,
    grid=(_NB,),
    in_specs=[
        pl.BlockSpec((_BN, _D), lambda i: (i, 0)),
        pl.BlockSpec((_D, 2 * _D), lambda i: (0, 0)),
        pl.BlockSpec((_D,), lambda i: (0,)),
    ],
    out_specs=[
        pl.BlockSpec((_BN, 128), lambda i: (i, 0)),
        pl.BlockSpec((_BN, 128), lambda i: (i, 0)),
        pl.BlockSpec((_BN, _D), lambda i: (i, 0)),
    ],
    out_shape=[
        jax.ShapeDtypeStruct((_N, 128), _f32),
        jax.ShapeDtypeStruct((_N, 128), _f32),
        jax.ShapeDtypeStruct((_N, _D), _f32),
    ],
)


def _tc_mid_body(a0_ref, a1_ref, cnt_ref, yrb_ref, w_ref, b_ref,
                 yl0_ref, yl1_ref, yrb2_ref):
    inv = 1.0 / jnp.maximum(cnt_ref[...][:, :1], 1.0)
    aggcat = jnp.concatenate([a0_ref[...], a1_ref[...]], axis=1)
    nf1 = jnp.maximum(aggcat * inv + yrb_ref[...], 0.0)
    r = jnp.dot(nf1, w_ref[...], preferred_element_type=_f32)
    yl0_ref[...] = r[:, :128]
    yl1_ref[...] = r[:, 128:256]
    yrb2_ref[...] = r[:, 256:] + b_ref[...][None, :]


_tc_mid = pl.pallas_call(
    _tc_mid_body,
    grid=(_NB,),
    in_specs=[
        pl.BlockSpec((_BN, 128), lambda i: (i, 0)),
        pl.BlockSpec((_BN, 128), lambda i: (i, 0)),
        pl.BlockSpec((_BN, 128), lambda i: (i, 0)),
        pl.BlockSpec((_BN, _D), lambda i: (i, 0)),
        pl.BlockSpec((_D, 2 * _D), lambda i: (0, 0)),
        pl.BlockSpec((_D,), lambda i: (0,)),
    ],
    out_specs=[
        pl.BlockSpec((_BN, 128), lambda i: (i, 0)),
        pl.BlockSpec((_BN, 128), lambda i: (i, 0)),
        pl.BlockSpec((_BN, _D), lambda i: (i, 0)),
    ],
    out_shape=[
        jax.ShapeDtypeStruct((_N, 128), _f32),
        jax.ShapeDtypeStruct((_N, 128), _f32),
        jax.ShapeDtypeStruct((_N, _D), _f32),
    ],
)


def _tc_nf2_body(a0_ref, a1_ref, cnt_ref, yrb_ref, nf2_ref, st_ref):
    i = pl.program_id(0)
    inv = 1.0 / jnp.maximum(cnt_ref[...][:, :1], 1.0)
    aggcat = jnp.concatenate([a0_ref[...], a1_ref[...]], axis=1)
    nf2 = aggcat * inv + yrb_ref[...]
    nf2_ref[...] = nf2
    blk = jnp.concatenate(
        [jnp.sum(nf2, axis=0)[None, :], jnp.sum(nf2 * nf2, axis=0)[None, :],
         jnp.zeros((6, _D), _f32)], axis=0)

    @pl.when(i == 0)
    def _():
        st_ref[...] = blk

    @pl.when(i > 0)
    def _():
        st_ref[...] += blk


_tc_nf2 = pl.pallas_call(
    _tc_nf2_body,
    grid=(_NB,),
    in_specs=[
        pl.BlockSpec((_BN, 128), lambda i: (i, 0)),
        pl.BlockSpec((_BN, 128), lambda i: (i, 0)),
        pl.BlockSpec((_BN, 128), lambda i: (i, 0)),
        pl.BlockSpec((_BN, _D), lambda i: (i, 0)),
    ],
    out_specs=[
        pl.BlockSpec((_BN, _D), lambda i: (i, 0)),
        pl.BlockSpec((8, _D), lambda i: (0, 0)),
    ],
    out_shape=[
        jax.ShapeDtypeStruct((_N, _D), _f32),
        jax.ShapeDtypeStruct((8, _D), _f32),
    ],
)


def _tc_head_body(nf2_ref, st_ref, rnd_ref, dg_ref, batch_ref, wf1t_ref,
                  aux_ref, gf_ref, ng_ref, sg_ref, kl_ref, pres_ref, p16_ref,
                  acc_ref, sca_ref):
    i = pl.program_id(0)
    nf2 = nf2_ref[...]
    af1 = jnp.tanh(jnp.dot(nf2, wf1t_ref[...], preferred_element_type=_f32)
                   + aux_ref[0, :][None, :])
    dl = jnp.sum(af1 * aux_ref[1, :][None, :], axis=1) + aux_ref[2, 0]
    p = 1.0 / (1.0 + jnp.exp(-dl))
    lp = 1.0 / (1.0 + jnp.exp(-(2.0 * p - 1.0 + dg_ref[...][:, 0])))
    ln = 1.0 - lp
    p16_ref[...] = jnp.broadcast_to(p[:, None], (_BN, 128))

    m = st_ref[0, :] / _N
    var = (st_ref[1, :] - _N * m * m) / (_N - 1)
    s = jnp.sqrt(jnp.maximum(var, 0.0))
    cmat = m[None, :] + rnd_ref[...] * s[None, :]

    oh = (batch_ref[...][:, :1]
          == lax.broadcasted_iota(jnp.int32, (_BN, _G), 1)).astype(_f32)
    dn = (((0,), (0,)), ((), ()))
    gfp = lax.dot_general(oh, nf2, dn, preferred_element_type=_f32)
    sgp = lax.dot_general(oh * lp[:, None], nf2, dn, preferred_element_type=_f32)
    ngp = sgp + lax.dot_general(oh * ln[:, None], cmat, dn,
                                preferred_element_type=_f32)

    lp2 = lp * lp
    arow = jnp.sum(lp2[:, None] * nf2, axis=0)
    brow = jnp.sum(lp2[:, None] * nf2 * nf2, axis=0)
    blk = jnp.concatenate([arow[None, :], brow[None, :],
                           jnp.zeros((6, _D), _f32)], axis=0)
    sln2 = jnp.sum(ln * ln)
    c0 = jnp.sum(lp2)
    npres = jnp.sum((p > 0.5).astype(_f32))

    @pl.when(i == 0)
    def _():
        gf_ref[...] = gfp
        sg_ref[...] = sgp
        ng_ref[...] = ngp
        acc_ref[...] = blk
        sca_ref[0] = sln2
        sca_ref[1] = c0
        sca_ref[2] = npres
        kl_ref[...] = jnp.zeros((1, 1), _f32)
        pres_ref[...] = jnp.zeros((1, 1), _f32)

    @pl.when(i > 0)
    def _():
        gf_ref[...] += gfp
        sg_ref[...] += sgp
        ng_ref[...] += ngp
        acc_ref[...] += blk
        sca_ref[0] += sln2
        sca_ref[1] += c0
        sca_ref[2] += npres

    @pl.when(i == _NB - 1)
    def _():
        eps = 1e-6
        se2 = (s + eps) * (s + eps)
        t1 = 0.5 * sca_ref[0] * jnp.sum(s * s / se2) / (_N * _D)
        q = acc_ref[1, :] - 2.0 * m * acc_ref[0, :] + m * m * sca_ref[1]
        t2 = jnp.sum(q / se2) / _D
        kl_ref[...] = jnp.full((1, 1), t1 + t2, _f32)
        pres_ref[...] = jnp.full((1, 1), sca_ref[2] / _N, _f32)


_tc_head = pl.pallas_call(
    _tc_head_body,
    grid=(_NB,),
    in_specs=[
        pl.BlockSpec((_BN, _D), lambda i: (i, 0)),
        pl.BlockSpec((8, _D), lambda i: (0, 0)),
        pl.BlockSpec((_BN, _D), lambda i: (i, 0)),
        pl.BlockSpec((_BN, 8), lambda i: (i, 0)),
        pl.BlockSpec((_BN, 8), lambda i: (i, 0)),
        pl.BlockSpec((_D, 64), lambda i: (0, 0)),
        pl.BlockSpec((8, 64), lambda i: (0, 0)),
    ],
    out_specs=[
        pl.BlockSpec((_G, _D), lambda i: (0, 0)),
        pl.BlockSpec((_G, _D), lambda i: (0, 0)),
        pl.BlockSpec((_G, _D), lambda i: (0, 0)),
        pl.BlockSpec((1, 1), lambda i: (0, 0)),
        pl.BlockSpec((1, 1), lambda i: (0, 0)),
        pl.BlockSpec((_BN, 128), lambda i: (i, 0)),
    ],
    out_shape=[
        jax.ShapeDtypeStruct((_G, _D), _f32),
        jax.ShapeDtypeStruct((_G, _D), _f32),
        jax.ShapeDtypeStruct((_G, _D), _f32),
        jax.ShapeDtypeStruct((1, 1), _f32),
        jax.ShapeDtypeStruct((1, 1), _f32),
        jax.ShapeDtypeStruct((_N, 128), _f32),
    ],
    scratch_shapes=[
        pltpu.VMEM((8, _D), _f32),
        pltpu.SMEM((4,), _f32),
    ],
)


def _zero_rows(zsrc, acc, s):
    pltpu.sync_copy(zsrc, acc.at[pl.ds(s * _NZ, _NZ)])

    @pl.when(s == _NSUB - 1)
    def _():
        pltpu.sync_copy(zsrc.at[pl.ds(0, _NTAIL)],
                        acc.at[pl.ds(_NTOFF, _NTAIL)])


def _write_rows(acc, out, s):
    pltpu.sync_copy(acc.at[pl.ds(s * _NZ, _NZ)], out.at[pl.ds(s * _NZ, _NZ)])

    @pl.when(s == _NSUB - 1)
    def _():
        pltpu.sync_copy(acc.at[pl.ds(_NTOFF, _NTAIL)],
                        out.at[pl.ds(_NTOFF, _NTAIL)])


def _sc_conv_body(y0, y1, srch, dsth, z128, out0, out1,
                  srcv, dstv, rows, acc, sem):
    c = lax.axis_index("c")
    s = lax.axis_index("s")

    _zero_rows(z128, acc, s)
    plsc.subcore_barrier()

    def run(y_hbm):
        def body(k, carry):
            base = s * _EP + k * _CH
            pltpu.sync_copy(srch.at[pl.ds(base, _CH)], srcv)
            pltpu.sync_copy(dsth.at[pl.ds(base, _CH)], dstv)
            pltpu.async_copy(y_hbm.at[srcv], rows, sem).wait()
            pltpu.sync_copy(rows, acc.at[dstv], add=True)
            return carry
        lax.fori_loop(0, _EP // _CH, body, 0)

    @pl.when(c == 0)
    def _():
        run(y0)

    @pl.when(c == 1)
    def _():
        run(y1)

    plsc.subcore_barrier()

    @pl.when(c == 0)
    def _():
        _write_rows(acc, out0, s)

    @pl.when(c == 1)
    def _():
        _write_rows(acc, out1, s)


_sc_conv = pl.kernel(
    _sc_conv_body,
    out_type=(jax.ShapeDtypeStruct((_N, 128), _f32),
              jax.ShapeDtypeStruct((_N, 128), _f32)),
    mesh=plsc.VectorSubcoreMesh(core_axis_name="c", subcore_axis_name="s"),
    scratch_types=[
        pltpu.VMEM((_CH,), jnp.int32),
        pltpu.VMEM((_CH,), jnp.int32),
        pltpu.VMEM((_CH, 128), _f32),
        pltpu.VMEM_SHARED((_N, 128), _f32),
        pltpu.SemaphoreType.DMA,
    ],
)


def _sc_deg_body(srch, dsth, z128, ones128, outi, outo,
                 idxv, onesv, acc, sem):
    c = lax.axis_index("c")
    s = lax.axis_index("s")

    _zero_rows(z128, acc, s)
    pltpu.sync_copy(ones128, onesv)
    plsc.subcore_barrier()

    def run(idx_hbm):
        def body(k, carry):
            base = s * _EP + k * _CH
            pltpu.sync_copy(idx_hbm.at[pl.ds(base, _CH)], idxv)
            pltpu.sync_copy(onesv, acc.at[idxv], add=True)
            return carry
        lax.fori_loop(0, _EP // _CH, body, 0)

    @pl.when(c == 0)
    def _():
        run(dsth)

    @pl.when(c == 1)
    def _():
        run(srch)

    plsc.subcore_barrier()

    @pl.when(c == 0)
    def _():
        _write_rows(acc, outi, s)

    @pl.when(c == 1)
    def _():
        _write_rows(acc, outo, s)


_sc_deg = pl.kernel(
    _sc_deg_body,
    out_type=(jax.ShapeDtypeStruct((_N, 128), _f32),
              jax.ShapeDtypeStruct((_N, 128), _f32)),
    mesh=plsc.VectorSubcoreMesh(core_axis_name="c", subcore_axis_name="s"),
    scratch_types=[
        pltpu.VMEM((_CH,), jnp.int32),
        pltpu.VMEM((_CH, 128), _f32),
        pltpu.VMEM_SHARED((_N, 128), _f32),
        pltpu.SemaphoreType.DMA,
    ],
)


_EP2 = _E // (2 * _NSUB)
_CH2 = 40


def _sc_tp_body(p16h, srch, dsth, z128, t0, t1, idxg, idxs, rows, acc, sem):
    c = lax.axis_index("c")
    s = lax.axis_index("s")

    _zero_rows(z128, acc, s)
    plsc.subcore_barrier()

    base0 = (c * _NSUB + s) * _EP2

    def body(k, carry):
        base = base0 + k * _CH2
        pltpu.sync_copy(dsth.at[pl.ds(base, _CH2)], idxg)
        pltpu.sync_copy(srch.at[pl.ds(base, _CH2)], idxs)
        pltpu.async_copy(p16h.at[idxg], rows, sem).wait()
        pltpu.sync_copy(rows, acc.at[idxs], add=True)
        return carry

    lax.fori_loop(0, _EP2 // _CH2, body, 0)
    plsc.subcore_barrier()

    @pl.when(c == 0)
    def _():
        _write_rows(acc, t0, s)

    @pl.when(c == 1)
    def _():
        _write_rows(acc, t1, s)


_sc_tp = pl.kernel(
    _sc_tp_body,
    out_type=(jax.ShapeDtypeStruct((_N, 128), _f32),
              jax.ShapeDtypeStruct((_N, 128), _f32)),
    mesh=plsc.VectorSubcoreMesh(core_axis_name="c", subcore_axis_name="s"),
    scratch_types=[
        pltpu.VMEM((_CH2,), jnp.int32),
        pltpu.VMEM((_CH2,), jnp.int32),
        pltpu.VMEM((_CH2, 128), _f32),
        pltpu.VMEM_SHARED((_N, 128), _f32),
        pltpu.SemaphoreType.DMA,
    ],
)


def _tc_pos_body(p_ref, t0_ref, t1_ref, od_ref, pos_ref, sca_ref):
    i = pl.program_id(0)
    p = p_ref[...]
    t = t0_ref[...] + t1_ref[...]
    spp = jnp.sum(p * t) / 128.0
    spd = jnp.sum(t) / 128.0
    sps = jnp.sum(p * od_ref[...]) / 128.0

    @pl.when(i == 0)
    def _():
        sca_ref[0] = spp
        sca_ref[1] = spd
        sca_ref[2] = sps
        pos_ref[...] = jnp.zeros((1, 1), _f32)

    @pl.when(i > 0)
    def _():
        sca_ref[0] += spp
        sca_ref[1] += spd
        sca_ref[2] += sps

    @pl.when(i == _NB - 1)
    def _():
        s_pp = sca_ref[0]
        s_pd = sca_ref[1]
        s_ps = sca_ref[2]
        d0 = s_pp / s_ps
        d1 = (_E - s_ps - s_pd + s_pp) / (_E - s_ps)
        pos = ((d0 - 1.0) * (d0 - 1.0) + (d1 - 1.0) * (d1 - 1.0)) * 0.5
        pos_ref[...] = jnp.full((1, 1), pos, _f32)


_tc_pos = pl.pallas_call(
    _tc_pos_body,
    grid=(_NB,),
    in_specs=[
        pl.BlockSpec((_BN, 128), lambda i: (i, 0)),
        pl.BlockSpec((_BN, 128), lambda i: (i, 0)),
        pl.BlockSpec((_BN, 128), lambda i: (i, 0)),
        pl.BlockSpec((_BN, 128), lambda i: (i, 0)),
    ],
    out_specs=pl.BlockSpec((1, 1), lambda i: (0, 0)),
    out_shape=jax.ShapeDtypeStruct((1, 1), _f32),
    scratch_shapes=[pltpu.SMEM((4,), _f32)],
)


def kernel(x, edge_index, batch, Wl1, Wr1, b1, Wl2, Wr2, b2,
           Wf1, bf1, Wf2, bf2):
    src = edge_index[0]
    dst = edge_index[1]

    w1cat = jnp.concatenate([Wl1.T, Wr1.T], axis=1)
    w2cat = jnp.concatenate([Wl2.T, Wr2.T], axis=1)
    wf1t = Wf1.T
    aux = jnp.zeros((8, 64), _f32)
    aux = aux.at[0, :].set(bf1)
    aux = aux.at[1, :].set(Wf2[0] - Wf2[1])
    aux = aux.at[2, 0].set(bf2[0] - bf2[1])

    u = jax.random.uniform(jax.random.key(42), (_N, 2),
                           minval=1e-10, maxval=1.0)
    gum = -jnp.log(-jnp.log(u))
    dg8 = jnp.broadcast_to((gum[:, 0] - gum[:, 1])[:, None], (_N, 8))
    rnd = jax.random.uniform(jax.random.key(43), (_N, _D), dtype=_f32)
    batch8 = jnp.broadcast_to(batch[:, None], (_N, 8))

    z128 = jnp.zeros((_NZ, 128), _f32)
    ones128 = jnp.ones((_CH, 128), _f32)

    yl0, yl1, yrb1 = _tc_lin(x, w1cat, b1)
    indeg, odeg = _sc_deg(src, dst, z128, ones128)
    agg0, agg1 = _sc_conv(yl0, yl1, src, dst, z128)
    yl20, yl21, yrb2 = _tc_mid(agg0, agg1, indeg, yrb1, w2cat, b2)
    agg20, agg21 = _sc_conv(yl20, yl21, src, dst, z128)
    nf2, stats = _tc_nf2(agg20, agg21, indeg, yrb2)
    gf, ng, sg, kl, pres, p16 = _tc_head(nf2, stats, rnd, dg8, batch8,
                                         wf1t, aux)
    t0, t1 = _sc_tp(p16, src, dst, z128)
    pos = _tc_pos(p16, t0, t1, odeg)

    return (gf, ng, sg, pos.reshape(()), kl.reshape(()), pres.reshape(()))
